# baseline (speedup 1.0000x reference)
import functools

import jax
import jax.numpy as jnp
from jax import lax
from jax.experimental import pallas as pl
from jax.experimental.pallas import tpu as pltpu
from jax.experimental.pallas import tpu_sc as plsc

N = 10000
E = 320000
F_IN = 128
H = 64
C = 40
L = 6
BN_EPS = 1e-5

NC = 2
NS = 16
GROUP = 128
G = 80
EPT = G * GROUP
EPAD = NC * NS * EPT
ROWS_PT = N // NS
ZROWS = 125

BLK = 1000


def _prep_edges(idx, val):
    pad = EPAD - E
    src = jnp.pad(idx[0], (0, pad)).reshape(NC, NS, G, GROUP)
    dst = jnp.pad(idx[1], (0, pad)).reshape(NC, NS, G, GROUP)
    vb = lax.bitcast_convert_type(jnp.pad(val, (0, pad)),
                                  jnp.int32).reshape(NC, NS, G, GROUP)
    return jnp.stack([src, dst, vb], axis=3)


NSS = G // 2
NEB = 4


def _sc_body(sup_hbm, edges_hbm, out_hbm, *rest):
    rbig = rest[:2]
    ebufs = rest[2:2 + NEB]
    sup_sh = rest[2 + NEB]
    acc_sh = rest[3 + NEB]
    sems = rest[4 + NEB:]
    seq = sems[:NEB]
    sgq = sems[NEB:NEB + 2]
    ssq = sems[NEB + 2:NEB + 4]
    c = lax.axis_index("c")
    s = lax.axis_index("s")

    cp_sup = pltpu.async_copy(sup_hbm.at[pl.ds(s * ROWS_PT, ROWS_PT)],
                              sup_sh.at[pl.ds(s * ROWS_PT, ROWS_PT)], sgq[0])

    @pl.loop(0, 2 * GROUP)
    def _(r):
        for k4 in range(H // 16):
            rbig[0][r, pl.ds(k4 * 16, 16)] = jnp.zeros((16,), jnp.float32)

    for z in range(2):
        pltpu.sync_copy(
            rbig[0], acc_sh.at[pl.ds(s * ROWS_PT + z * 2 * GROUP, 2 * GROUP)])
    pltpu.sync_copy(rbig[0].at[pl.ds(0, ROWS_PT - 4 * GROUP)],
                    acc_sh.at[pl.ds(s * ROWS_PT + 4 * GROUP,
                                    ROWS_PT - 4 * GROUP)])

    cp_sup.wait()
    plsc.subcore_barrier()

    def compute(rb, eb):
        for hf in range(2):
            @pl.loop(0, GROUP, step=16)
            def _(e0):
                val16 = plsc.bitcast(eb[hf, 2, pl.ds(e0, 16)], jnp.float32)
                for t in range(16):
                    vv = jnp.full((16,), val16[t], jnp.float32)
                    for k4 in range(H // 16):
                        sl = pl.ds(k4 * 16, 16)
                        r = hf * GROUP + e0 + t
                        rb[r, sl] = rb[r, sl] * vv

    def fire_gathers(eslot, rb, sem):
        pltpu.async_copy(sup_sh.at[ebufs[eslot].at[0, 0]],
                         rb.at[pl.ds(0, GROUP)], sem)
        pltpu.async_copy(sup_sh.at[ebufs[eslot].at[1, 0]],
                         rb.at[pl.ds(GROUP, GROUP)], sem)

    def wait2(sem, rb):
        for hf in range(2):
            pltpu.make_async_copy(
                sup_sh.at[ebufs[0].at[0, 0]],
                rb.at[pl.ds(hf * GROUP, GROUP)], sem).wait()

    def fire_scatters(eslot, rb, sem):
        pltpu.async_copy(rb.at[pl.ds(0, GROUP)],
                         acc_sh.at[ebufs[eslot].at[0, 1]], sem, add=True)
        pltpu.async_copy(rb.at[pl.ds(GROUP, GROUP)],
                         acc_sh.at[ebufs[eslot].at[1, 1]], sem, add=True)

    pltpu.async_copy(edges_hbm.at[c, s, pl.ds(0, 2)], ebufs[0], seq[0])
    pltpu.async_copy(edges_hbm.at[c, s, pl.ds(2, 2)], ebufs[1], seq[1])
    pltpu.make_async_copy(edges_hbm.at[c, s, pl.ds(0, 2)],
                          ebufs[0], seq[0]).wait()
    fire_gathers(0, rbig[0], sgq[0])

    @pl.loop(0, NSS // 4)
    def _(p):
        for j in range(4):
            q = p * 4 + j
            u = j % 2
            v = 1 - u
            rb = rbig[u]
            eb = ebufs[j]

            @pl.when(q + 2 < NSS)
            def _():
                pltpu.async_copy(edges_hbm.at[c, s, pl.ds((q + 2) * 2, 2)],
                                 ebufs[(j + 2) % 4], seq[(j + 2) % 4])

            wait2(sgq[u], rb)

            @pl.when(q + 1 < NSS)
            def _():
                @pl.when(q >= 1)
                def _():
                    for hf in range(2):
                        pltpu.make_async_copy(
                            rbig[v].at[pl.ds(hf * GROUP, GROUP)],
                            acc_sh.at[ebufs[0].at[0, 1]], ssq[v]).wait()

                pltpu.make_async_copy(edges_hbm.at[c, s, pl.ds(0, 2)],
                                      ebufs[(j + 1) % 4],
                                      seq[(j + 1) % 4]).wait()
                fire_gathers((j + 1) % 4, rbig[v], sgq[v])

            compute(rb, eb)
            fire_scatters(j, rb, ssq[u])

    for u in range(2):
        for hf in range(2):
            pltpu.make_async_copy(
                rbig[u].at[pl.ds(hf * GROUP, GROUP)],
                acc_sh.at[ebufs[0].at[0, 1]], ssq[u]).wait()

    plsc.subcore_barrier()

    pltpu.sync_copy(acc_sh.at[pl.ds(s * ROWS_PT, ROWS_PT)],
                    out_hbm.at[c, pl.ds(s * ROWS_PT, ROWS_PT)])


def _spmm_sc(support, edges):
    kfn = pl.kernel(
        _sc_body,
        out_type=jax.ShapeDtypeStruct((NC, N, H), jnp.float32),
        mesh=plsc.VectorSubcoreMesh(core_axis_name="c", subcore_axis_name="s"),
        compiler_params=pltpu.CompilerParams(use_tc_tiling_on_sc=False,
                                             needs_layout_passes=False),
        scratch_types=(
            [pltpu.VMEM((2 * GROUP, H), jnp.float32)] * 2
            + [pltpu.VMEM((2, 3, GROUP), jnp.int32)] * NEB
            + [pltpu.VMEM_SHARED((N, H), jnp.float32),
               pltpu.VMEM_SHARED((N, H), jnp.float32)]
            + [pltpu.SemaphoreType.DMA] * (NEB + 4)
        ),
    )
    return kfn(support, edges)


def _tc_in(x, W_in, b_in8, W0):
    def body(x_ref, wi_ref, bi_ref, w0_ref, o_ref):
        h = jnp.dot(x_ref[...], wi_ref[...], preferred_element_type=jnp.float32)
        h = h + bi_ref[0:1, :]
        o_ref[...] = jnp.dot(h, w0_ref[...], preferred_element_type=jnp.float32)

    return pl.pallas_call(
        body,
        grid=(N // BLK,),
        in_specs=[
            pl.BlockSpec((BLK, F_IN), lambda i: (i, 0)),
            pl.BlockSpec((F_IN, H), lambda i: (0, 0)),
            pl.BlockSpec((8, H), lambda i: (0, 0)),
            pl.BlockSpec((H, H), lambda i: (0, 0)),
        ],
        out_specs=pl.BlockSpec((BLK, H), lambda i: (i, 0)),
        out_shape=jax.ShapeDtypeStruct((N, H), jnp.float32),
    )(x, W_in, b_in8, W0)


def _tc_mid(parts, ss8, W):
    def body(p_ref, ss_ref, w_ref, o_ref):
        agg = p_ref[0] + p_ref[1]
        h = jax.nn.relu(agg * ss_ref[0:1, :] + ss_ref[1:2, :])
        o_ref[...] = jnp.dot(h, w_ref[...], preferred_element_type=jnp.float32)

    return pl.pallas_call(
        body,
        grid=(N // BLK,),
        in_specs=[
            pl.BlockSpec((NC, BLK, H), lambda i: (0, i, 0)),
            pl.BlockSpec((8, H), lambda i: (0, 0)),
            pl.BlockSpec((H, H), lambda i: (0, 0)),
        ],
        out_specs=pl.BlockSpec((BLK, H), lambda i: (i, 0)),
        out_shape=jax.ShapeDtypeStruct((N, H), jnp.float32),
    )(parts, ss8, W)


def _tc_out(parts, ss8, W_out, b_out8):
    def body(p_ref, ss_ref, w_ref, bo_ref, o_ref):
        agg = p_ref[0] + p_ref[1]
        h = jax.nn.relu(agg * ss_ref[0:1, :] + ss_ref[1:2, :])
        logits = jnp.dot(h, w_ref[...], preferred_element_type=jnp.float32)
        logits = logits + bo_ref[0:1, :]
        m = jnp.max(logits, axis=1, keepdims=True)
        shifted = logits - m
        lse = jnp.log(jnp.sum(jnp.exp(shifted), axis=1, keepdims=True))
        o_ref[...] = shifted - lse

    return pl.pallas_call(
        body,
        grid=(N // BLK,),
        in_specs=[
            pl.BlockSpec((NC, BLK, H), lambda i: (0, i, 0)),
            pl.BlockSpec((8, H), lambda i: (0, 0)),
            pl.BlockSpec((H, C), lambda i: (0, 0)),
            pl.BlockSpec((8, C), lambda i: (0, 0)),
        ],
        out_specs=pl.BlockSpec((BLK, C), lambda i: (i, 0)),
        out_shape=jax.ShapeDtypeStruct((N, C), jnp.float32),
    )(parts, ss8, W_out, b_out8)


def _pad8(v):
    return jnp.zeros((8, v.shape[0]), v.dtype).at[0].set(v)


def kernel(x, sample1_adj_indices, sample1_adj_values,
           sample2_adj_indices, sample2_adj_values,
           W_in, b_in, W_conv, b_conv, gamma, beta, W_out, b_out):
    inv_std = 1.0 / jnp.sqrt(1.0 + BN_EPS)
    scale = inv_std * gamma
    shift = b_conv * scale + beta

    e1 = _prep_edges(sample1_adj_indices, sample1_adj_values)
    e2 = _prep_edges(sample2_adj_indices, sample2_adj_values)

    support = _tc_in(x, W_in, _pad8(b_in), W_conv[0])
    for i in range(L):
        edges = e1 if i < L // 2 else e2
        ss8 = jnp.concatenate(
            [scale[i:i + 1], shift[i:i + 1], jnp.zeros((6, H), jnp.float32)], axis=0)
        parts = _spmm_sc(support, edges)
        if i < L - 1:
            support = _tc_mid(parts, ss8, W_conv[i + 1])
        else:
            return _tc_out(parts, ss8, W_out, _pad8(b_out))


# speedup vs baseline: 1.1405x; 1.1405x over previous
import functools

import jax
import jax.numpy as jnp
from jax import lax
from jax.experimental import pallas as pl
from jax.experimental.pallas import tpu as pltpu
from jax.experimental.pallas import tpu_sc as plsc

N = 10000
E = 320000
F_IN = 128
H = 64
C = 40
L = 6
BN_EPS = 1e-5

NC = 2
NS = 16
GROUP = 128
G = 80
EPT = G * GROUP
EPAD = NC * NS * EPT
ROWS_PT = N // NS
ZROWS = 125

BLK = 1000


def _prep_edges(idx, val):
    pad = EPAD - E
    src = jnp.pad(idx[0], (0, pad)).reshape(NC, NS, G, GROUP)
    dst = jnp.pad(idx[1], (0, pad)).reshape(NC, NS, G, GROUP)
    vb = lax.bitcast_convert_type(jnp.pad(val, (0, pad)),
                                  jnp.int32).reshape(NC, NS, G, GROUP)
    return jnp.stack([src, dst, vb], axis=3)


NB = 5
LEAD = 2


def _sc_body(sup_hbm, edges_hbm, out_hbm, *rest):
    rbufs = rest[:NB]
    ebufs = rest[NB:2 * NB]
    sup_sh = rest[2 * NB]
    acc_sh = rest[2 * NB + 1]
    sems = rest[2 * NB + 2:]
    sg = sems[:NB]
    ss = sems[NB:2 * NB]
    se = sems[2 * NB:]
    c = lax.axis_index("c")
    s = lax.axis_index("s")

    cp_sup = pltpu.async_copy(sup_hbm.at[pl.ds(s * ROWS_PT, ROWS_PT)],
                              sup_sh.at[pl.ds(s * ROWS_PT, ROWS_PT)], sg[0])

    @pl.loop(0, GROUP)
    def _(r):
        for k4 in range(H // 16):
            rbufs[0][r, pl.ds(k4 * 16, 16)] = jnp.zeros((16,), jnp.float32)

    for z in range(4):
        pltpu.sync_copy(rbufs[0],
                        acc_sh.at[pl.ds(s * ROWS_PT + z * GROUP, GROUP)])
    pltpu.sync_copy(rbufs[0].at[pl.ds(0, ROWS_PT - 4 * GROUP)],
                    acc_sh.at[pl.ds(s * ROWS_PT + 4 * GROUP,
                                    ROWS_PT - 4 * GROUP)])

    cp_sup.wait()
    plsc.subcore_barrier()

    def compute(rb, eb):
        @pl.loop(0, GROUP, step=16)
        def _(e0):
            val16 = plsc.bitcast(eb[2, pl.ds(e0, 16)], jnp.float32)
            for t in range(16):
                vv = jnp.full((16,), val16[t], jnp.float32)
                for k4 in range(H // 16):
                    sl = pl.ds(k4 * 16, 16)
                    rb[e0 + t, sl] = rb[e0 + t, sl] * vv

    for j in range(LEAD):
        pltpu.async_copy(edges_hbm.at[c, s, j], ebufs[j], se[j])
    for j in range(LEAD):
        pltpu.make_async_copy(edges_hbm.at[c, s, j], ebufs[j], se[j]).wait()
        pltpu.async_copy(sup_sh.at[ebufs[j].at[0]], rbufs[j], sg[j])

    @pl.loop(0, G // NB)
    def _(p):
        for j in range(NB):
            g = p * NB + j
            rb = rbufs[j]
            eb = ebufs[j]
            jn = (j + LEAD) % NB
            gn = g + LEAD

            pltpu.make_async_copy(sup_sh.at[eb.at[0]], rb, sg[j]).wait()

            @pl.when(gn < G)
            def _():
                @pl.when(gn >= NB)
                def _():
                    pltpu.make_async_copy(
                        rbufs[jn], acc_sh.at[eb.at[1]], ss[jn]).wait()

                pltpu.async_copy(edges_hbm.at[c, s, gn], ebufs[jn], se[jn])

            compute(rb, eb)
            pltpu.async_copy(rb, acc_sh.at[eb.at[1]], ss[j], add=True)

            @pl.when(gn < G)
            def _():
                pltpu.make_async_copy(
                    edges_hbm.at[c, s, 0], ebufs[jn], se[jn]).wait()
                pltpu.async_copy(sup_sh.at[ebufs[jn].at[0]], rbufs[jn], sg[jn])

    for j in range(NB):
        pltpu.make_async_copy(rbufs[j], acc_sh.at[ebufs[j].at[1]], ss[j]).wait()

    plsc.subcore_barrier()

    pltpu.sync_copy(acc_sh.at[pl.ds(s * ROWS_PT, ROWS_PT)],
                    out_hbm.at[c, pl.ds(s * ROWS_PT, ROWS_PT)])


def _spmm_sc(support, edges):
    kfn = pl.kernel(
        _sc_body,
        out_type=jax.ShapeDtypeStruct((NC, N, H), jnp.float32),
        mesh=plsc.VectorSubcoreMesh(core_axis_name="c", subcore_axis_name="s"),
        compiler_params=pltpu.CompilerParams(use_tc_tiling_on_sc=False,
                                             needs_layout_passes=False),
        scratch_types=(
            [pltpu.VMEM((GROUP, H), jnp.float32)] * NB
            + [pltpu.VMEM((3, GROUP), jnp.int32)] * NB
            + [pltpu.VMEM_SHARED((N, H), jnp.float32),
               pltpu.VMEM_SHARED((N, H), jnp.float32)]
            + [pltpu.SemaphoreType.DMA] * (3 * NB)
        ),
    )
    return kfn(support, edges)


def _tc_in(x, W_in, b_in8, W0):
    def body(x_ref, wi_ref, bi_ref, w0_ref, o_ref):
        h = jnp.dot(x_ref[...], wi_ref[...], preferred_element_type=jnp.float32)
        h = h + bi_ref[0:1, :]
        o_ref[...] = jnp.dot(h, w0_ref[...], preferred_element_type=jnp.float32)

    return pl.pallas_call(
        body,
        grid=(N // BLK,),
        in_specs=[
            pl.BlockSpec((BLK, F_IN), lambda i: (i, 0)),
            pl.BlockSpec((F_IN, H), lambda i: (0, 0)),
            pl.BlockSpec((8, H), lambda i: (0, 0)),
            pl.BlockSpec((H, H), lambda i: (0, 0)),
        ],
        out_specs=pl.BlockSpec((BLK, H), lambda i: (i, 0)),
        out_shape=jax.ShapeDtypeStruct((N, H), jnp.float32),
    )(x, W_in, b_in8, W0)


def _tc_mid(parts, ss8, W):
    def body(p_ref, ss_ref, w_ref, o_ref):
        agg = p_ref[0] + p_ref[1]
        h = jax.nn.relu(agg * ss_ref[0:1, :] + ss_ref[1:2, :])
        o_ref[...] = jnp.dot(h, w_ref[...], preferred_element_type=jnp.float32)

    return pl.pallas_call(
        body,
        grid=(N // BLK,),
        in_specs=[
            pl.BlockSpec((NC, BLK, H), lambda i: (0, i, 0)),
            pl.BlockSpec((8, H), lambda i: (0, 0)),
            pl.BlockSpec((H, H), lambda i: (0, 0)),
        ],
        out_specs=pl.BlockSpec((BLK, H), lambda i: (i, 0)),
        out_shape=jax.ShapeDtypeStruct((N, H), jnp.float32),
    )(parts, ss8, W)


def _tc_out(parts, ss8, W_out, b_out8):
    def body(p_ref, ss_ref, w_ref, bo_ref, o_ref):
        agg = p_ref[0] + p_ref[1]
        h = jax.nn.relu(agg * ss_ref[0:1, :] + ss_ref[1:2, :])
        logits = jnp.dot(h, w_ref[...], preferred_element_type=jnp.float32)
        logits = logits + bo_ref[0:1, :]
        m = jnp.max(logits, axis=1, keepdims=True)
        shifted = logits - m
        lse = jnp.log(jnp.sum(jnp.exp(shifted), axis=1, keepdims=True))
        o_ref[...] = shifted - lse

    return pl.pallas_call(
        body,
        grid=(N // BLK,),
        in_specs=[
            pl.BlockSpec((NC, BLK, H), lambda i: (0, i, 0)),
            pl.BlockSpec((8, H), lambda i: (0, 0)),
            pl.BlockSpec((H, C), lambda i: (0, 0)),
            pl.BlockSpec((8, C), lambda i: (0, 0)),
        ],
        out_specs=pl.BlockSpec((BLK, C), lambda i: (i, 0)),
        out_shape=jax.ShapeDtypeStruct((N, C), jnp.float32),
    )(parts, ss8, W_out, b_out8)


def _pad8(v):
    return jnp.zeros((8, v.shape[0]), v.dtype).at[0].set(v)


def kernel(x, sample1_adj_indices, sample1_adj_values,
           sample2_adj_indices, sample2_adj_values,
           W_in, b_in, W_conv, b_conv, gamma, beta, W_out, b_out):
    inv_std = 1.0 / jnp.sqrt(1.0 + BN_EPS)
    scale = inv_std * gamma
    shift = b_conv * scale + beta

    e1 = _prep_edges(sample1_adj_indices, sample1_adj_values)
    e2 = _prep_edges(sample2_adj_indices, sample2_adj_values)

    support = _tc_in(x, W_in, _pad8(b_in), W_conv[0])
    for i in range(L):
        edges = e1 if i < L // 2 else e2
        ss8 = jnp.concatenate(
            [scale[i:i + 1], shift[i:i + 1], jnp.zeros((6, H), jnp.float32)], axis=0)
        parts = _spmm_sc(support, edges)
        if i < L - 1:
            support = _tc_mid(parts, ss8, W_conv[i + 1])
        else:
            return _tc_out(parts, ss8, W_out, _pad8(b_out))


# speedup vs baseline: 2.0821x; 1.8256x over previous
import functools

import jax
import jax.numpy as jnp
from jax import lax
from jax.experimental import pallas as pl
from jax.experimental.pallas import tpu as pltpu
from jax.experimental.pallas import tpu_sc as plsc

N = 10000
E = 320000
F_IN = 128
H = 64
C = 40
L = 6
BN_EPS = 1e-5

NC = 2
NS = 16
GROUP = 128
G = 80
EPT = G * GROUP
EPAD = NC * NS * EPT
ROWS_PT = N // NS
ZROWS = 125

BLK = 1000


def _prep_edges(idx, val):
    pad = EPAD - E
    src = jnp.pad(idx[0], (0, pad)).reshape(NC, NS, G, GROUP)
    dst = jnp.pad(idx[1], (0, pad)).reshape(NC, NS, G, GROUP)
    vb = lax.bitcast_convert_type(jnp.pad(val, (0, pad)),
                                  jnp.int32).reshape(NC, NS, G, GROUP)
    return jnp.stack([src, dst, vb], axis=3)


NB = 5
LEAD = 2


def _sc_body(sup_hbm, edges_hbm, out_hbm, *rest):
    rbufs = rest[:NB]
    ebufs = rest[NB:2 * NB]
    sup_sh = rest[2 * NB]
    acc_sh = rest[2 * NB + 1]
    sems = rest[2 * NB + 2:]
    sg = sems[:NB]
    ss = sems[NB:2 * NB]
    se = sems[2 * NB:]
    c = lax.axis_index("c")
    s = lax.axis_index("s")

    cp_sup = pltpu.async_copy(sup_hbm.at[pl.ds(s * ROWS_PT, ROWS_PT)],
                              sup_sh.at[pl.ds(s * ROWS_PT, ROWS_PT)], sg[0])

    @pl.loop(0, GROUP)
    def _(r):
        for k4 in range(H // 16):
            rbufs[0][r, pl.ds(k4 * 16, 16)] = jnp.zeros((16,), jnp.float32)

    for z in range(4):
        pltpu.sync_copy(rbufs[0],
                        acc_sh.at[pl.ds(s * ROWS_PT + z * GROUP, GROUP)])
    pltpu.sync_copy(rbufs[0].at[pl.ds(0, ROWS_PT - 4 * GROUP)],
                    acc_sh.at[pl.ds(s * ROWS_PT + 4 * GROUP,
                                    ROWS_PT - 4 * GROUP)])

    cp_sup.wait()
    plsc.subcore_barrier()

    def compute(rb, eb):
        @plsc.parallel_loop(0, GROUP, step=16, unroll=2)
        def _(e0):
            val16 = plsc.bitcast(eb[2, pl.ds(e0, 16)], jnp.float32)
            for t in range(16):
                vv = jnp.full((16,), val16[t], jnp.float32)
                for k4 in range(H // 16):
                    sl = pl.ds(k4 * 16, 16)
                    rb[e0 + t, sl] = rb[e0 + t, sl] * vv

    for j in range(LEAD):
        pltpu.async_copy(edges_hbm.at[c, s, j], ebufs[j], se[j])
    for j in range(LEAD):
        pltpu.make_async_copy(edges_hbm.at[c, s, j], ebufs[j], se[j]).wait()
        pltpu.async_copy(sup_sh.at[ebufs[j].at[0]], rbufs[j], sg[j])

    @pl.loop(0, G // NB)
    def _(p):
        for j in range(NB):
            g = p * NB + j
            rb = rbufs[j]
            eb = ebufs[j]
            jn = (j + LEAD) % NB
            gn = g + LEAD

            pltpu.make_async_copy(sup_sh.at[eb.at[0]], rb, sg[j]).wait()

            @pl.when(gn < G)
            def _():
                @pl.when(gn >= NB)
                def _():
                    pltpu.make_async_copy(
                        rbufs[jn], acc_sh.at[eb.at[1]], ss[jn]).wait()

                pltpu.async_copy(edges_hbm.at[c, s, gn], ebufs[jn], se[jn])

            compute(rb, eb)
            pltpu.async_copy(rb, acc_sh.at[eb.at[1]], ss[j], add=True)

            @pl.when(gn < G)
            def _():
                pltpu.make_async_copy(
                    edges_hbm.at[c, s, 0], ebufs[jn], se[jn]).wait()
                pltpu.async_copy(sup_sh.at[ebufs[jn].at[0]], rbufs[jn], sg[jn])

    for j in range(NB):
        pltpu.make_async_copy(rbufs[j], acc_sh.at[ebufs[j].at[1]], ss[j]).wait()

    plsc.subcore_barrier()

    pltpu.sync_copy(acc_sh.at[pl.ds(s * ROWS_PT, ROWS_PT)],
                    out_hbm.at[c, pl.ds(s * ROWS_PT, ROWS_PT)])


def _spmm_sc(support, edges):
    kfn = pl.kernel(
        _sc_body,
        out_type=jax.ShapeDtypeStruct((NC, N, H), jnp.float32),
        mesh=plsc.VectorSubcoreMesh(core_axis_name="c", subcore_axis_name="s"),
        compiler_params=pltpu.CompilerParams(use_tc_tiling_on_sc=False,
                                             needs_layout_passes=False),
        scratch_types=(
            [pltpu.VMEM((GROUP, H), jnp.float32)] * NB
            + [pltpu.VMEM((3, GROUP), jnp.int32)] * NB
            + [pltpu.VMEM_SHARED((N, H), jnp.float32),
               pltpu.VMEM_SHARED((N, H), jnp.float32)]
            + [pltpu.SemaphoreType.DMA] * (3 * NB)
        ),
    )
    return kfn(support, edges)


def _tc_in(x, W_in, b_in8, W0):
    def body(x_ref, wi_ref, bi_ref, w0_ref, o_ref):
        h = jnp.dot(x_ref[...], wi_ref[...], preferred_element_type=jnp.float32)
        h = h + bi_ref[0:1, :]
        o_ref[...] = jnp.dot(h, w0_ref[...], preferred_element_type=jnp.float32)

    return pl.pallas_call(
        body,
        grid=(N // BLK,),
        in_specs=[
            pl.BlockSpec((BLK, F_IN), lambda i: (i, 0)),
            pl.BlockSpec((F_IN, H), lambda i: (0, 0)),
            pl.BlockSpec((8, H), lambda i: (0, 0)),
            pl.BlockSpec((H, H), lambda i: (0, 0)),
        ],
        out_specs=pl.BlockSpec((BLK, H), lambda i: (i, 0)),
        out_shape=jax.ShapeDtypeStruct((N, H), jnp.float32),
    )(x, W_in, b_in8, W0)


def _tc_mid(parts, ss8, W):
    def body(p_ref, ss_ref, w_ref, o_ref):
        agg = p_ref[0] + p_ref[1]
        h = jax.nn.relu(agg * ss_ref[0:1, :] + ss_ref[1:2, :])
        o_ref[...] = jnp.dot(h, w_ref[...], preferred_element_type=jnp.float32)

    return pl.pallas_call(
        body,
        grid=(N // BLK,),
        in_specs=[
            pl.BlockSpec((NC, BLK, H), lambda i: (0, i, 0)),
            pl.BlockSpec((8, H), lambda i: (0, 0)),
            pl.BlockSpec((H, H), lambda i: (0, 0)),
        ],
        out_specs=pl.BlockSpec((BLK, H), lambda i: (i, 0)),
        out_shape=jax.ShapeDtypeStruct((N, H), jnp.float32),
    )(parts, ss8, W)


def _tc_out(parts, ss8, W_out, b_out8):
    def body(p_ref, ss_ref, w_ref, bo_ref, o_ref):
        agg = p_ref[0] + p_ref[1]
        h = jax.nn.relu(agg * ss_ref[0:1, :] + ss_ref[1:2, :])
        logits = jnp.dot(h, w_ref[...], preferred_element_type=jnp.float32)
        logits = logits + bo_ref[0:1, :]
        m = jnp.max(logits, axis=1, keepdims=True)
        shifted = logits - m
        lse = jnp.log(jnp.sum(jnp.exp(shifted), axis=1, keepdims=True))
        o_ref[...] = shifted - lse

    return pl.pallas_call(
        body,
        grid=(N // BLK,),
        in_specs=[
            pl.BlockSpec((NC, BLK, H), lambda i: (0, i, 0)),
            pl.BlockSpec((8, H), lambda i: (0, 0)),
            pl.BlockSpec((H, C), lambda i: (0, 0)),
            pl.BlockSpec((8, C), lambda i: (0, 0)),
        ],
        out_specs=pl.BlockSpec((BLK, C), lambda i: (i, 0)),
        out_shape=jax.ShapeDtypeStruct((N, C), jnp.float32),
    )(parts, ss8, W_out, b_out8)


def _pad8(v):
    return jnp.zeros((8, v.shape[0]), v.dtype).at[0].set(v)


def kernel(x, sample1_adj_indices, sample1_adj_values,
           sample2_adj_indices, sample2_adj_values,
           W_in, b_in, W_conv, b_conv, gamma, beta, W_out, b_out):
    inv_std = 1.0 / jnp.sqrt(1.0 + BN_EPS)
    scale = inv_std * gamma
    shift = b_conv * scale + beta

    e1 = _prep_edges(sample1_adj_indices, sample1_adj_values)
    e2 = _prep_edges(sample2_adj_indices, sample2_adj_values)

    support = _tc_in(x, W_in, _pad8(b_in), W_conv[0])
    for i in range(L):
        edges = e1 if i < L // 2 else e2
        ss8 = jnp.concatenate(
            [scale[i:i + 1], shift[i:i + 1], jnp.zeros((6, H), jnp.float32)], axis=0)
        parts = _spmm_sc(support, edges)
        if i < L - 1:
            support = _tc_mid(parts, ss8, W_conv[i + 1])
        else:
            return _tc_out(parts, ss8, W_out, _pad8(b_out))
